# baseline (speedup 1.0000x reference)
import jax
import jax.numpy as jnp
from jax import lax
from jax.experimental import pallas as pl
from jax.experimental.pallas import tpu as pltpu
from jax.experimental.pallas import tpu_sc as plsc

GAMMA_F = 10.0
LN2 = 0.6931471805599453

NC, NS, L = 2, 16, 16
NW = NC * NS

B = 4
K = 16
CB = 8
N_NETS = 100000
P_PINS = 400000

NETS_PER_W = 3125
GSTRIDE = 120
GCOUNT = 27
SLAB_B = 3256

PINS_PER_W = 12500
CHUNKS_A = 98
WIN_A = CHUNKS_A * 128

CHUNKS_A0 = 26
WIN_A0 = CHUNKS_A0 * 128


def _fast_log(s):
    bits = lax.bitcast_convert_type(s, jnp.int32)
    e = (bits >> 23) - 127
    m = lax.bitcast_convert_type((bits & 0x007FFFFF) | 0x3F800000, jnp.float32)
    z = (m - 1.0) / (m + 1.0)
    z2 = z * z
    p = jnp.float32(1.0 / 7.0)
    p = p * z2 + jnp.float32(1.0 / 5.0)
    p = p * z2 + jnp.float32(1.0 / 3.0)
    p = p * z2 + jnp.float32(1.0)
    return e.astype(jnp.float32) * jnp.float32(LN2) + (2.0 * z) * p


def _worker_id():
    return lax.axis_index("s") * NC + lax.axis_index("c")


def _win_start(per_w, win, total):
    w = _worker_id()
    return jnp.minimum((w * per_w) // 8 * 8, total - win)


def _stage_a0_body(pos_pl, pos_t, vbuf, obuf, sem_i, sem_o):
    start0 = _win_start(N_NETS // NW, WIN_A0, N_NETS)

    iot = lax.iota(jnp.int32, L)
    rowv = [iot + jnp.int32(16 * i) for i in range(8)]
    csplat = [jnp.full((L,), c, jnp.int32) for c in range(8)]

    def issue(j, s):
        m0 = start0 + j * 128
        for c in range(CB):
            pltpu.async_copy(pos_pl.at[c, pl.ds(m0, 128)],
                             vbuf.at[s, c], sem_i.at[s])

    def wait_in(s):
        for c in range(CB):
            pltpu.make_async_copy(pos_pl.at[0, pl.ds(0, 128)],
                                  vbuf.at[s, c], sem_i.at[s]).wait()

    for s in range(2):
        issue(jnp.int32(s), s)

    def chunk(i, carry, s):
        j = i * 2 + s
        wait_in(s)
        @pl.when(j >= 2)
        def _():
            pltpu.make_async_copy(obuf.at[s], pos_t.at[pl.ds(0, 128)],
                                  sem_o.at[s]).wait()
        for i8 in range(8):
            for c in range(CB):
                v = plsc.load_gather(vbuf.at[s], [csplat[c], rowv[i8]])
                plsc.store_scatter(obuf.at[s], [rowv[i8], csplat[c]],
                                   v * jnp.float32(GAMMA_F))
        m0 = start0 + j * 128
        pltpu.async_copy(obuf.at[s], pos_t.at[pl.ds(m0, 128)], sem_o.at[s])
        @pl.when(j + 2 < CHUNKS_A0)
        def _():
            issue(j + 2, s)
        return carry

    def outer(i, carry):
        carry = chunk(i, carry, 0)
        carry = chunk(i, carry, 1)
        return carry

    lax.fori_loop(0, CHUNKS_A0 // 2, outer, jnp.int32(0))
    for s in range(2):
        pltpu.make_async_copy(obuf.at[s], pos_t.at[pl.ds(0, 128)],
                              sem_o.at[s]).wait()


def _stage_a_body(pos_t, pin_to_macro, offs_pl, pin_pos, idx_v, gbuf, obuf,
                  wbuf, sem_g, sem_o, sem_w):
    start_p = _win_start(PINS_PER_W, WIN_A, P_PINS)
    pltpu.sync_copy(pin_to_macro.at[pl.ds(start_p, WIN_A)], idx_v)

    iot = lax.iota(jnp.int32, L)
    rowv = [iot + jnp.int32(16 * i) for i in range(8)]
    csplat = [jnp.full((L,), c, jnp.int32) for c in range(8)]

    def issue(j, s):
        base = start_p + j * 128
        pltpu.async_copy(pos_t.at[idx_v.at[pl.ds(j * 128, 128)]],
                         gbuf.at[s], sem_g.at[s])
        for d in range(2):
            pltpu.async_copy(offs_pl.at[d, pl.ds(base, 128)],
                             obuf.at[s, d], sem_o.at[s])

    def wait_in(s):
        pltpu.make_async_copy(pos_t.at[pl.ds(0, 128)], gbuf.at[s],
                              sem_g.at[s]).wait()
        for d in range(2):
            pltpu.make_async_copy(offs_pl.at[0, pl.ds(0, 128)],
                                  obuf.at[s, d], sem_o.at[s]).wait()

    for s in range(2):
        issue(jnp.int32(s), s)

    def chunk(i, carry, s):
        j = i * 2 + s
        wait_in(s)
        @pl.when(j >= 2)
        def _():
            pltpu.make_async_copy(wbuf.at[s], pin_pos.at[pl.ds(0, 128)],
                                  sem_w.at[s]).wait()
        for i8 in range(8):
            offx = obuf[s, 0, pl.ds(16 * i8, 16)] * jnp.float32(GAMMA_F)
            offy = obuf[s, 1, pl.ds(16 * i8, 16)] * jnp.float32(GAMMA_F)
            for c in range(CB):
                pv = plsc.load_gather(gbuf.at[s], [rowv[i8], csplat[c]])
                sm = pv + (offx if c % 2 == 0 else offy)
                plsc.store_scatter(wbuf.at[s], [rowv[i8], csplat[c]], sm)
        base = start_p + j * 128
        pltpu.async_copy(wbuf.at[s], pin_pos.at[pl.ds(base, 128)], sem_w.at[s])
        @pl.when(j + 2 < CHUNKS_A)
        def _():
            issue(j + 2, s)
        return carry

    def outer(i, carry):
        carry = chunk(i, carry, 0)
        carry = chunk(i, carry, 1)
        return carry

    lax.fori_loop(0, CHUNKS_A // 2, outer, jnp.int32(0))
    for s in range(2):
        pltpu.make_async_copy(wbuf.at[s], pin_pos.at[pl.ds(0, 128)],
                              sem_w.at[s]).wait()


def _stage_b_body(pin_pos, ntp_pl, out, idx_v, gbuf, outv, sem_g):
    w = _worker_id()
    start_n = jnp.minimum((w * NETS_PER_W) // 8 * 8, N_NETS - SLAB_B)
    s_off = w * NETS_PER_W - start_n
    pltpu.sync_copy(ntp_pl.at[:, pl.ds(start_n, SLAB_B)], idx_v)

    iot = lax.iota(jnp.int32, L)
    rowk = [iot + jnp.int32(k * 128) for k in range(K)]
    csplat = [jnp.full((L,), c, jnp.int32) for c in range(CB)]

    def rstart(g):
        return jnp.minimum((s_off + GSTRIDE * g) // 8 * 8, SLAB_B - 128)

    def issue(g, s):
        r0 = rstart(g)
        for k in range(K):
            pltpu.async_copy(pin_pos.at[idx_v.at[k, pl.ds(r0, 128)]],
                             gbuf.at[s, pl.ds(k * 128, 128)], sem_g.at[s])

    def wait_g(s):
        for k in range(K):
            pltpu.make_async_copy(pin_pos.at[pl.ds(0, 128)],
                                  gbuf.at[s, pl.ds(k * 128, 128)],
                                  sem_g.at[s]).wait()

    for s in range(3):
        issue(jnp.int32(s), s)

    def group(g, accs, s):
        wait_g(s)
        nbase0 = rstart(g) - s_off
        lo = GSTRIDE * g
        hi = jnp.minimum(lo + GSTRIDE, NETS_PER_W)
        zero = jnp.zeros((L,), jnp.float32)

        def lane_grp(lg, accs):
            nbase = nbase0 + lg * 16 + iot
            valid = (nbase >= lo) & (nbase < hi)
            accs = list(accs)
            rows = [rowk[k] + lg * 16 for k in range(K)]
            for c in range(CB):
                wv = [plsc.load_gather(gbuf.at[s], [rows[k], csplat[c]])
                      for k in range(K)]
                wm = wv
                while len(wm) > 1:
                    wm = [jnp.maximum(wm[2 * t], wm[2 * t + 1])
                          for t in range(len(wm) // 2)]
                wn = wv
                while len(wn) > 1:
                    wn = [jnp.minimum(wn[2 * t], wn[2 * t + 1])
                          for t in range(len(wn) // 2)]
                w_max, w_min = wm[0], wn[0]
                ep = [jnp.exp(v - w_max) for v in wv]
                en = [jnp.exp(w_min - v) for v in wv]
                while len(ep) > 1:
                    ep = [ep[2 * t] + ep[2 * t + 1]
                          for t in range(len(ep) // 2)]
                while len(en) > 1:
                    en = [en[2 * t] + en[2 * t + 1]
                          for t in range(len(en) // 2)]
                wl = (w_max - w_min + _fast_log(ep[0] * en[0])) \
                    * jnp.float32(1.0 / GAMMA_F)
                accs[c // 2] = accs[c // 2] + jnp.where(valid, wl, zero)
            return tuple(accs)

        accs = lax.fori_loop(0, 8, lane_grp, accs)
        @pl.when(g + 3 < GCOUNT)
        def _():
            issue(g + 3, s)
        return accs

    def outer(i, accs):
        accs = group(i * 3, accs, 0)
        accs = group(i * 3 + 1, accs, 1)
        accs = group(i * 3 + 2, accs, 2)
        return accs

    zero = jnp.zeros((L,), jnp.float32)
    accs = lax.fori_loop(0, GCOUNT // 3, outer, (zero, zero, zero, zero))
    for b in range(B):
        outv[b, :] = accs[b]
    pltpu.sync_copy(outv, out.at[w])


_MESH = plsc.VectorSubcoreMesh(core_axis_name="c", subcore_axis_name="s",
                               num_cores=NC, num_subcores=NS)
_PARAMS = pltpu.CompilerParams(needs_layout_passes=False,
                               use_tc_tiling_on_sc=False)

_stage_a0 = pl.kernel(
    _stage_a0_body,
    out_type=jax.ShapeDtypeStruct((N_NETS, CB), jnp.float32),
    mesh=_MESH,
    compiler_params=_PARAMS,
    scratch_types=[
        pltpu.VMEM((2, CB, 128), jnp.float32),
        pltpu.VMEM((2, 128, CB), jnp.float32),
        pltpu.SemaphoreType.DMA((2,)),
        pltpu.SemaphoreType.DMA((2,)),
    ],
)

_stage_a = pl.kernel(
    _stage_a_body,
    out_type=jax.ShapeDtypeStruct((P_PINS, CB), jnp.float32),
    mesh=_MESH,
    compiler_params=_PARAMS,
    scratch_types=[
        pltpu.VMEM((WIN_A,), jnp.int32),
        pltpu.VMEM((2, 128, CB), jnp.float32),
        pltpu.VMEM((2, 2, 128), jnp.float32),
        pltpu.VMEM((2, 128, CB), jnp.float32),
        pltpu.SemaphoreType.DMA((2,)),
        pltpu.SemaphoreType.DMA((2,)),
        pltpu.SemaphoreType.DMA((2,)),
    ],
)

_stage_b = pl.kernel(
    _stage_b_body,
    out_type=jax.ShapeDtypeStruct((NW, B, L), jnp.float32),
    mesh=_MESH,
    compiler_params=_PARAMS,
    scratch_types=[
        pltpu.VMEM((K, SLAB_B), jnp.int32),
        pltpu.VMEM((3, K * 128, CB), jnp.float32),
        pltpu.VMEM((B, L), jnp.float32),
        pltpu.SemaphoreType.DMA((3,)),
    ],
)


@jax.jit
def kernel(positions, net_to_pin, pin_to_macro, pin_offsets):
    pos_pl = jnp.transpose(positions, (0, 2, 1)).reshape(CB, N_NETS)
    offs_pl = jnp.transpose(pin_offsets, (1, 0))
    ntp_pl = jnp.transpose(net_to_pin.astype(jnp.int32), (1, 0))
    ptm = pin_to_macro.astype(jnp.int32)

    pos_t = _stage_a0(pos_pl)
    pin_pos = _stage_a(pos_t, ptm, offs_pl)
    partial = _stage_b(pin_pos, ntp_pl)
    return partial.sum(axis=(0, 2))


# speedup vs baseline: 1.0322x; 1.0322x over previous
import jax
import jax.numpy as jnp
from jax import lax
from jax.experimental import pallas as pl
from jax.experimental.pallas import tpu as pltpu
from jax.experimental.pallas import tpu_sc as plsc

GAMMA_F = 10.0
LN2 = 0.6931471805599453

NC, NS, L = 2, 16, 16
NW = NC * NS

B = 4
K = 16
CB = 8
N_NETS = 100000
P_PINS = 400000

NETS_PER_W = 3125
GSTRIDE = 121
GCOUNT = 26
SLAB_B = 3256

PINS_PER_W = 12500
CHUNKS_A = 98
WIN_A = CHUNKS_A * 128

CHUNKS_A0 = 26
WIN_A0 = CHUNKS_A0 * 128


def _fast_log(s):
    bits = lax.bitcast_convert_type(s, jnp.int32)
    e = (bits >> 23) - 127
    m = lax.bitcast_convert_type((bits & 0x007FFFFF) | 0x3F800000, jnp.float32)
    z = (m - 1.0) / (m + 1.0)
    z2 = z * z
    p = jnp.float32(1.0 / 7.0)
    p = p * z2 + jnp.float32(1.0 / 5.0)
    p = p * z2 + jnp.float32(1.0 / 3.0)
    p = p * z2 + jnp.float32(1.0)
    return e.astype(jnp.float32) * jnp.float32(LN2) + (2.0 * z) * p


def _worker_id():
    return lax.axis_index("s") * NC + lax.axis_index("c")


def _win_start(per_w, win, total):
    w = _worker_id()
    return jnp.minimum((w * per_w) // 8 * 8, total - win)


def _stage_a0_body(pos_pl, pos_t, vbuf, obuf, sem_i, sem_o):
    start0 = _win_start(N_NETS // NW, WIN_A0, N_NETS)

    iot = lax.iota(jnp.int32, L)
    rowv = [iot + jnp.int32(16 * i) for i in range(8)]
    csplat = [jnp.full((L,), c, jnp.int32) for c in range(8)]

    def issue(j, s):
        m0 = start0 + j * 128
        for c in range(CB):
            pltpu.async_copy(pos_pl.at[c, pl.ds(m0, 128)],
                             vbuf.at[s, c], sem_i.at[s])

    def wait_in(s):
        for c in range(CB):
            pltpu.make_async_copy(pos_pl.at[0, pl.ds(0, 128)],
                                  vbuf.at[s, c], sem_i.at[s]).wait()

    for s in range(2):
        issue(jnp.int32(s), s)

    def chunk(i, carry, s):
        j = i * 2 + s
        wait_in(s)
        @pl.when(j >= 2)
        def _():
            pltpu.make_async_copy(obuf.at[s], pos_t.at[pl.ds(0, 128)],
                                  sem_o.at[s]).wait()
        for i8 in range(8):
            for c in range(CB):
                v = plsc.load_gather(vbuf.at[s], [csplat[c], rowv[i8]])
                plsc.store_scatter(obuf.at[s], [rowv[i8], csplat[c]],
                                   v * jnp.float32(GAMMA_F))
        m0 = start0 + j * 128
        pltpu.async_copy(obuf.at[s], pos_t.at[pl.ds(m0, 128)], sem_o.at[s])
        @pl.when(j + 2 < CHUNKS_A0)
        def _():
            issue(j + 2, s)
        return carry

    def outer(i, carry):
        carry = chunk(i, carry, 0)
        carry = chunk(i, carry, 1)
        return carry

    lax.fori_loop(0, CHUNKS_A0 // 2, outer, jnp.int32(0))
    for s in range(2):
        pltpu.make_async_copy(obuf.at[s], pos_t.at[pl.ds(0, 128)],
                              sem_o.at[s]).wait()


def _stage_a_body(pos_t, pin_to_macro, offs_pl, pin_pos, idx_v, gbuf, obuf,
                  wbuf, sem_g, sem_o, sem_w):
    start_p = _win_start(PINS_PER_W, WIN_A, P_PINS)
    pltpu.sync_copy(pin_to_macro.at[pl.ds(start_p, WIN_A)], idx_v)

    iot = lax.iota(jnp.int32, L)
    rowv = [iot + jnp.int32(16 * i) for i in range(8)]
    csplat = [jnp.full((L,), c, jnp.int32) for c in range(8)]

    def issue(j, s):
        base = start_p + j * 128
        pltpu.async_copy(pos_t.at[idx_v.at[pl.ds(j * 128, 128)]],
                         gbuf.at[s], sem_g.at[s])
        for d in range(2):
            pltpu.async_copy(offs_pl.at[d, pl.ds(base, 128)],
                             obuf.at[s, d], sem_o.at[s])

    def wait_in(s):
        pltpu.make_async_copy(pos_t.at[pl.ds(0, 128)], gbuf.at[s],
                              sem_g.at[s]).wait()
        for d in range(2):
            pltpu.make_async_copy(offs_pl.at[0, pl.ds(0, 128)],
                                  obuf.at[s, d], sem_o.at[s]).wait()

    for s in range(2):
        issue(jnp.int32(s), s)

    def chunk(i, carry, s):
        j = i * 2 + s
        wait_in(s)
        @pl.when(j >= 2)
        def _():
            pltpu.make_async_copy(wbuf.at[s], pin_pos.at[pl.ds(0, 128)],
                                  sem_w.at[s]).wait()
        for i8 in range(8):
            offx = obuf[s, 0, pl.ds(16 * i8, 16)] * jnp.float32(GAMMA_F)
            offy = obuf[s, 1, pl.ds(16 * i8, 16)] * jnp.float32(GAMMA_F)
            for c in range(CB):
                pv = plsc.load_gather(gbuf.at[s], [rowv[i8], csplat[c]])
                sm = pv + (offx if c % 2 == 0 else offy)
                plsc.store_scatter(wbuf.at[s], [rowv[i8], csplat[c]], sm)
        base = start_p + j * 128
        pltpu.async_copy(wbuf.at[s], pin_pos.at[pl.ds(base, 128)], sem_w.at[s])
        @pl.when(j + 2 < CHUNKS_A)
        def _():
            issue(j + 2, s)
        return carry

    def outer(i, carry):
        carry = chunk(i, carry, 0)
        carry = chunk(i, carry, 1)
        return carry

    lax.fori_loop(0, CHUNKS_A // 2, outer, jnp.int32(0))
    for s in range(2):
        pltpu.make_async_copy(wbuf.at[s], pin_pos.at[pl.ds(0, 128)],
                              sem_w.at[s]).wait()


def _stage_b_body(pin_pos, ntp_pl, out, idx_v, gbuf, outv, sem_g):
    w = _worker_id()
    start_n = jnp.minimum((w * NETS_PER_W) // 8 * 8, N_NETS - SLAB_B)
    s_off = w * NETS_PER_W - start_n
    pltpu.sync_copy(ntp_pl.at[:, pl.ds(start_n, SLAB_B)], idx_v)

    iot = lax.iota(jnp.int32, L)
    rowk = [iot + jnp.int32(k * 128) for k in range(K)]
    csplat = [jnp.full((L,), c, jnp.int32) for c in range(CB)]

    def rstart(g):
        return jnp.minimum((s_off + GSTRIDE * g) // 8 * 8, SLAB_B - 128)

    def issue(g, s):
        r0 = rstart(g)
        for k in range(K):
            pltpu.async_copy(pin_pos.at[idx_v.at[k, pl.ds(r0, 128)]],
                             gbuf.at[s, pl.ds(k * 128, 128)], sem_g.at[s])

    def wait_g(s):
        for k in range(K):
            pltpu.make_async_copy(pin_pos.at[pl.ds(0, 128)],
                                  gbuf.at[s, pl.ds(k * 128, 128)],
                                  sem_g.at[s]).wait()

    for s in range(2):
        issue(jnp.int32(s), s)

    def group(g, accs, s):
        wait_g(s)
        nbase0 = rstart(g) - s_off
        lo = GSTRIDE * g
        hi = jnp.minimum(lo + GSTRIDE, NETS_PER_W)
        zero = jnp.zeros((L,), jnp.float32)

        def lane_grp(lg, accs):
            nbase = nbase0 + lg * 16 + iot
            valid = (nbase >= lo) & (nbase < hi)
            accs = list(accs)
            rows = [rowk[k] + lg * 16 for k in range(K)]
            for c in range(CB):
                wv = [plsc.load_gather(gbuf.at[s], [rows[k], csplat[c]])
                      for k in range(K)]
                wm = wv
                while len(wm) > 1:
                    wm = [jnp.maximum(wm[2 * t], wm[2 * t + 1])
                          for t in range(len(wm) // 2)]
                wn = wv
                while len(wn) > 1:
                    wn = [jnp.minimum(wn[2 * t], wn[2 * t + 1])
                          for t in range(len(wn) // 2)]
                w_max, w_min = wm[0], wn[0]
                ep = [jnp.exp(v - w_max) for v in wv]
                en = [jnp.exp(w_min - v) for v in wv]
                while len(ep) > 1:
                    ep = [ep[2 * t] + ep[2 * t + 1]
                          for t in range(len(ep) // 2)]
                while len(en) > 1:
                    en = [en[2 * t] + en[2 * t + 1]
                          for t in range(len(en) // 2)]
                wl = (w_max - w_min + _fast_log(ep[0] * en[0])) \
                    * jnp.float32(1.0 / GAMMA_F)
                accs[c // 2] = accs[c // 2] + jnp.where(valid, wl, zero)
            return tuple(accs)

        accs = lax.fori_loop(0, 8, lane_grp, accs)
        @pl.when(g + 2 < GCOUNT)
        def _():
            issue(g + 2, s)
        return accs

    def outer(i, accs):
        accs = group(i * 2, accs, 0)
        accs = group(i * 2 + 1, accs, 1)
        return accs

    zero = jnp.zeros((L,), jnp.float32)
    accs = lax.fori_loop(0, GCOUNT // 2, outer, (zero, zero, zero, zero))
    for b in range(B):
        outv[b, :] = accs[b]
    pltpu.sync_copy(outv, out.at[w])


_MESH = plsc.VectorSubcoreMesh(core_axis_name="c", subcore_axis_name="s",
                               num_cores=NC, num_subcores=NS)
_PARAMS = pltpu.CompilerParams(needs_layout_passes=False,
                               use_tc_tiling_on_sc=False)

_stage_a0 = pl.kernel(
    _stage_a0_body,
    out_type=jax.ShapeDtypeStruct((N_NETS, CB), jnp.float32),
    mesh=_MESH,
    compiler_params=_PARAMS,
    scratch_types=[
        pltpu.VMEM((2, CB, 128), jnp.float32),
        pltpu.VMEM((2, 128, CB), jnp.float32),
        pltpu.SemaphoreType.DMA((2,)),
        pltpu.SemaphoreType.DMA((2,)),
    ],
)

_stage_a = pl.kernel(
    _stage_a_body,
    out_type=jax.ShapeDtypeStruct((P_PINS, CB), jnp.float32),
    mesh=_MESH,
    compiler_params=_PARAMS,
    scratch_types=[
        pltpu.VMEM((WIN_A,), jnp.int32),
        pltpu.VMEM((2, 128, CB), jnp.float32),
        pltpu.VMEM((2, 2, 128), jnp.float32),
        pltpu.VMEM((2, 128, CB), jnp.float32),
        pltpu.SemaphoreType.DMA((2,)),
        pltpu.SemaphoreType.DMA((2,)),
        pltpu.SemaphoreType.DMA((2,)),
    ],
)

_stage_b = pl.kernel(
    _stage_b_body,
    out_type=jax.ShapeDtypeStruct((NW, B, L), jnp.float32),
    mesh=_MESH,
    compiler_params=_PARAMS,
    scratch_types=[
        pltpu.VMEM((K, SLAB_B), jnp.int32),
        pltpu.VMEM((2, K * 128, CB), jnp.float32),
        pltpu.VMEM((B, L), jnp.float32),
        pltpu.SemaphoreType.DMA((2,)),
    ],
)


@jax.jit
def kernel(positions, net_to_pin, pin_to_macro, pin_offsets):
    pos_pl = jnp.transpose(positions, (0, 2, 1)).reshape(CB, N_NETS)
    offs_pl = jnp.transpose(pin_offsets, (1, 0))
    ntp_pl = jnp.transpose(net_to_pin.astype(jnp.int32), (1, 0))
    ptm = pin_to_macro.astype(jnp.int32)

    pos_t = _stage_a0(pos_pl)
    pin_pos = _stage_a(pos_t, ptm, offs_pl)
    partial = _stage_b(pin_pos, ntp_pl)
    return partial.sum(axis=(0, 2))
